# baseline (speedup 1.0000x reference)
import functools

import jax
import jax.numpy as jnp
from jax import lax
from jax.experimental import pallas as pl
from jax.experimental.pallas import tpu as pltpu
from jax.experimental.pallas import tpu_sc as plsc

_B = 16384
_D = 16
_NC = 2
_NS = 16
_NW = _NC * _NS
_BPW = _B // _NW
_CHUNK = 128
_NIDX = 3 * _BPW
_NCHUNK = _NIDX // _CHUNK
_NQ = 4

_LN2 = 0.6931471805599453
_SQRT2 = 1.4142135623730951


def _log16(x):
    xi = lax.bitcast_convert_type(x, jnp.int32)
    e = jnp.right_shift(xi, 23) - 127
    m = lax.bitcast_convert_type(
        jnp.bitwise_or(jnp.bitwise_and(xi, 0x007FFFFF), 0x3F800000), jnp.float32)
    big = m > _SQRT2
    m = jnp.where(big, m * 0.5, m)
    ef = e.astype(jnp.float32) + jnp.where(big, 1.0, 0.0)
    z = (m - 1.0) / (m + 1.0)
    z2 = z * z
    p = z * (2.0 + z2 * (0.66666667 + z2 * (0.4 + z2 * 0.28571429)))
    return ef * _LN2 + p


_mesh = plsc.VectorSubcoreMesh(core_axis_name="c", subcore_axis_name="s")


@functools.partial(
    pl.kernel,
    mesh=_mesh,
    compiler_params=pltpu.CompilerParams(
        needs_layout_passes=False, use_tc_tiling_on_sc=True),
    out_type=jax.ShapeDtypeStruct((_B,), jnp.float32),
    scratch_types=[
        pltpu.VMEM((_NCHUNK, _CHUNK), jnp.int32),
        pltpu.VMEM((_NCHUNK, _CHUNK), jnp.int32),
        pltpu.VMEM((_NCHUNK, _CHUNK), jnp.int32),
        pltpu.VMEM((2, 3, _CHUNK, _CHUNK), jnp.float32),
        pltpu.VMEM((_BPW,), jnp.float32),
        pltpu.SemaphoreType.DMA,
        pltpu.SemaphoreType.DMA,
    ],
)
def _tste_sc(idx_hbm, table_hbm, out_hbm, idx_v, rowid_v, raw_v, bufs_v,
             out_v, sem0, sem1):
    wid = lax.axis_index("s") * _NC + lax.axis_index("c")

    pltpu.sync_copy(idx_hbm.at[wid], idx_v)

    lane = lax.iota(jnp.int32, 16)

    for comp in range(3):
        for q in range(_NQ):
            c = comp * _NQ + q
            for s in range(8):
                t = q * 128 + s * 16 + lane
                p = 3 * t + comp
                v = plsc.load_gather(
                    idx_v, [jnp.right_shift(p, 7), jnp.bitwise_and(p, 127)])
                raw_v.at[c][pl.ds(s * 16, 16)] = v
                rowid_v.at[c][pl.ds(s * 16, 16)] = jnp.right_shift(v, 3)

    sems = [sem0, sem1]

    def fire(q):
        return [
            pltpu.async_copy(
                table_hbm.at[rowid_v.at[comp * _NQ + q]],
                bufs_v.at[q % 2, comp],
                sems[q % 2],
            )
            for comp in range(3)
        ]

    pending = fire(0)
    for q in range(_NQ):
        for cp in pending:
            cp.wait()
        if q + 1 < _NQ:
            pending = fire(q + 1)

        buf_h = bufs_v.at[q % 2, 0]
        buf_w = bufs_v.at[q % 2, 1]
        buf_l = bufs_v.at[q % 2, 2]
        raw_h = raw_v.at[0 * _NQ + q]
        raw_w = raw_v.at[1 * _NQ + q]
        raw_l = raw_v.at[2 * _NQ + q]

        def group(s, carry, buf_h=buf_h, buf_w=buf_w, buf_l=buf_l,
                  raw_h=raw_h, raw_w=raw_w, raw_l=raw_l, q=q):
            rowloc = s * 16 + lane
            cb_h = jnp.bitwise_and(raw_h[pl.ds(s * 16, 16)], 7) * 16
            cb_w = jnp.bitwise_and(raw_w[pl.ds(s * 16, 16)], 7) * 16
            cb_l = jnp.bitwise_and(raw_l[pl.ds(s * 16, 16)], 7) * 16
            accw = jnp.zeros((16,), jnp.float32)
            accl = jnp.zeros((16,), jnp.float32)
            for d in range(_D):
                hd = plsc.load_gather(buf_h, [rowloc, cb_h + d])
                wd = plsc.load_gather(buf_w, [rowloc, cb_w + d])
                ld = plsc.load_gather(buf_l, [rowloc, cb_l + d])
                dw = hd - wd
                dl = hd - ld
                accw = accw + dw * dw
                accl = accl + dl * dl
            x = 1.0 + (1.0 + accw) / (1.0 + accl)
            out_v[pl.ds(q * 128 + s * 16, 16)] = _log16(x)
            return carry

        lax.fori_loop(0, 8, group, 0)

    base = pl.multiple_of(wid * _BPW, 8)
    pltpu.sync_copy(out_v, out_hbm.at[pl.ds(base, _BPW)])


def kernel(h_w_l, embedding):
    idx = h_w_l.reshape(_NW, _NCHUNK, _CHUNK)
    table2 = embedding.reshape(125000, 128)
    return _tste_sc(idx, table2)


# speedup vs baseline: 1.2936x; 1.2936x over previous
import functools

import jax
import jax.numpy as jnp
from jax import lax
from jax.experimental import pallas as pl
from jax.experimental.pallas import tpu as pltpu
from jax.experimental.pallas import tpu_sc as plsc

_B = 16384
_D = 16
_NC = 2
_NS = 16
_NW = _NC * _NS
_BPW = _B // _NW
_NIDX = 3 * _BPW
_NPH = 32
_EPP = _NIDX // _NPH
_TPP = _BPW // _NPH

_LN2 = 0.6931471805599453
_SQRT2 = 1.4142135623730951


def _log16(x):
    xi = lax.bitcast_convert_type(x, jnp.int32)
    e = jnp.right_shift(xi, 23) - 127
    m = lax.bitcast_convert_type(
        jnp.bitwise_or(jnp.bitwise_and(xi, 0x007FFFFF), 0x3F800000), jnp.float32)
    big = m > _SQRT2
    m = jnp.where(big, m * 0.5, m)
    ef = e.astype(jnp.float32) + jnp.where(big, 1.0, 0.0)
    z = (m - 1.0) / (m + 1.0)
    z2 = z * z
    p = z * (2.0 + z2 * (0.66666667 + z2 * (0.4 + z2 * 0.28571429)))
    return ef * _LN2 + p


_mesh = plsc.VectorSubcoreMesh(core_axis_name="c", subcore_axis_name="s")


@functools.partial(
    pl.kernel,
    mesh=_mesh,
    compiler_params=pltpu.CompilerParams(
        needs_layout_passes=False, use_tc_tiling_on_sc=True),
    out_type=jax.ShapeDtypeStruct((_B,), jnp.float32),
    scratch_types=[
        pltpu.VMEM((_NIDX,), jnp.int32),
        pltpu.VMEM((_EPP, 8, _D), jnp.float32),
        pltpu.VMEM((_EPP, 8, _D), jnp.float32),
        pltpu.VMEM((_BPW,), jnp.float32),
        pltpu.SemaphoreType.DMA,
        pltpu.SemaphoreType.DMA,
    ],
)
def _tste_sc(idx_hbm, table_hbm, out_hbm, idx_v, rows_a, rows_b, out_v,
             sem0, sem1):
    wid = lax.axis_index("s") * _NC + lax.axis_index("c")

    pltpu.sync_copy(idx_hbm.at[wid], idx_v)

    lane = lax.iota(jnp.int32, 16)

    def fire(p, buf, sem):
        for g in range(_EPP // 16):
            vec = idx_v[pl.ds(p * _EPP + g * 16, 16)]
            for i in range(16):
                tb = pl.multiple_of(jnp.right_shift(vec[i], 3) * 8, 8)
                pltpu.async_copy(
                    table_hbm.at[pl.ds(tb, 8), :], buf.at[g * 16 + i], sem)

    def drain(buf, sem):
        for k in range(_EPP):
            pltpu.make_async_copy(
                table_hbm.at[pl.ds(0, 8), :], buf.at[k], sem).wait()

    def compute(p, buf):
        e_h = p * _EPP + 3 * lane
        th = plsc.load_gather(idx_v, [e_h])
        tw = plsc.load_gather(idx_v, [e_h + 1])
        tl = plsc.load_gather(idx_v, [e_h + 2])
        sub_h = jnp.bitwise_and(th, 7)
        sub_w = jnp.bitwise_and(tw, 7)
        sub_l = jnp.bitwise_and(tl, 7)
        l_h = 3 * lane
        accw = jnp.zeros((16,), jnp.float32)
        accl = jnp.zeros((16,), jnp.float32)
        for d in range(_D):
            dv = jnp.full((16,), d, jnp.int32)
            hd = plsc.load_gather(buf, [l_h, sub_h, dv])
            wd = plsc.load_gather(buf, [l_h + 1, sub_w, dv])
            ld = plsc.load_gather(buf, [l_h + 2, sub_l, dv])
            dw = hd - wd
            dl = hd - ld
            accw = accw + dw * dw
            accl = accl + dl * dl
        x = 1.0 + (1.0 + accw) / (1.0 + accl)
        out_v[pl.ds(p * _TPP, 16)] = _log16(x)

    fire(0, rows_a, sem0)
    fire(1, rows_b, sem1)

    def pair(j, carry):
        p0 = 2 * j

        drain(rows_a, sem0)
        compute(p0, rows_a)

        @pl.when(p0 + 2 < _NPH)
        def _():
            fire(p0 + 2, rows_a, sem0)

        drain(rows_b, sem1)
        compute(p0 + 1, rows_b)

        @pl.when(p0 + 3 < _NPH)
        def _():
            fire(p0 + 3, rows_b, sem1)

        return carry

    lax.fori_loop(0, _NPH // 2, pair, 0)

    base = pl.multiple_of(wid * _BPW, 8)
    pltpu.sync_copy(out_v, out_hbm.at[pl.ds(base, _BPW)])


def kernel(h_w_l, embedding):
    idx = h_w_l.reshape(_NW, _NIDX)
    return _tste_sc(idx, embedding)


# speedup vs baseline: 1.2948x; 1.0009x over previous
import functools

import jax
import jax.numpy as jnp
from jax import lax
from jax.experimental import pallas as pl
from jax.experimental.pallas import tpu as pltpu
from jax.experimental.pallas import tpu_sc as plsc

_B = 16384
_D = 16
_NC = 2
_NS = 16
_NW = _NC * _NS
_BPW = _B // _NW
_NIDX = 3 * _BPW
_NPH = 32
_EPP = _NIDX // _NPH
_TPP = _BPW // _NPH

_LN2 = 0.6931471805599453
_SQRT2 = 1.4142135623730951


def _log16(x):
    xi = lax.bitcast_convert_type(x, jnp.int32)
    e = jnp.right_shift(xi, 23) - 127
    m = lax.bitcast_convert_type(
        jnp.bitwise_or(jnp.bitwise_and(xi, 0x007FFFFF), 0x3F800000), jnp.float32)
    big = m > _SQRT2
    m = jnp.where(big, m * 0.5, m)
    ef = e.astype(jnp.float32) + jnp.where(big, 1.0, 0.0)
    z = (m - 1.0) / (m + 1.0)
    z2 = z * z
    p = z * (2.0 + z2 * (0.66666667 + z2 * (0.4 + z2 * 0.28571429)))
    return ef * _LN2 + p


_mesh = plsc.VectorSubcoreMesh(core_axis_name="c", subcore_axis_name="s")


@functools.partial(
    pl.kernel,
    mesh=_mesh,
    compiler_params=pltpu.CompilerParams(
        needs_layout_passes=False, use_tc_tiling_on_sc=True),
    out_type=jax.ShapeDtypeStruct((_B,), jnp.float32),
    scratch_types=[
        pltpu.VMEM((_NIDX,), jnp.int32),
        pltpu.VMEM((_EPP, 8, _D), jnp.float32),
        pltpu.VMEM((_EPP, 8, _D), jnp.float32),
        pltpu.VMEM((_BPW,), jnp.float32),
        pltpu.SemaphoreType.DMA,
        pltpu.SemaphoreType.DMA,
    ],
)
def _tste_sc(idx_hbm, table_hbm, out_hbm, idx_v, rows_a, rows_b, out_v,
             sem0, sem1):
    wid = lax.axis_index("s") * _NC + lax.axis_index("c")

    pltpu.sync_copy(idx_hbm.at[wid], idx_v)

    lane = lax.iota(jnp.int32, 16)

    def fire(p, buf, sem):
        for g in range(_EPP // 16):
            vec = idx_v[pl.ds(p * _EPP + g * 16, 16)]
            for i in range(16):
                tb = pl.multiple_of(jnp.right_shift(vec[i], 3) * 8, 8)
                pltpu.async_copy(
                    table_hbm.at[pl.ds(tb, 8), :], buf.at[g * 16 + i], sem)

    def drain(buf, sem):
        pltpu.make_async_copy(
            table_hbm.at[pl.ds(0, _EPP * 8), :].reshape(_EPP, 8, _D),
            buf, sem).wait()

    def compute(p, buf):
        e_h = p * _EPP + 3 * lane
        th = plsc.load_gather(idx_v, [e_h])
        tw = plsc.load_gather(idx_v, [e_h + 1])
        tl = plsc.load_gather(idx_v, [e_h + 2])
        sub_h = jnp.bitwise_and(th, 7)
        sub_w = jnp.bitwise_and(tw, 7)
        sub_l = jnp.bitwise_and(tl, 7)
        l_h = 3 * lane
        accw = jnp.zeros((16,), jnp.float32)
        accl = jnp.zeros((16,), jnp.float32)
        for d in range(_D):
            dv = jnp.full((16,), d, jnp.int32)
            hd = plsc.load_gather(buf, [l_h, sub_h, dv])
            wd = plsc.load_gather(buf, [l_h + 1, sub_w, dv])
            ld = plsc.load_gather(buf, [l_h + 2, sub_l, dv])
            dw = hd - wd
            dl = hd - ld
            accw = accw + dw * dw
            accl = accl + dl * dl
        x = 1.0 + (1.0 + accw) / (1.0 + accl)
        out_v[pl.ds(p * _TPP, 16)] = _log16(x)

    fire(0, rows_a, sem0)
    fire(1, rows_b, sem1)

    def pair(j, carry):
        p0 = 2 * j

        drain(rows_a, sem0)
        compute(p0, rows_a)

        @pl.when(p0 + 2 < _NPH)
        def _():
            fire(p0 + 2, rows_a, sem0)

        drain(rows_b, sem1)
        compute(p0 + 1, rows_b)

        @pl.when(p0 + 3 < _NPH)
        def _():
            fire(p0 + 3, rows_b, sem1)

        return carry

    lax.fori_loop(0, _NPH // 2, pair, 0)

    base = pl.multiple_of(wid * _BPW, 8)
    pltpu.sync_copy(out_v, out_hbm.at[pl.ds(base, _BPW)])


def kernel(h_w_l, embedding):
    idx = h_w_l.reshape(_NW, _NIDX)
    return _tste_sc(idx, embedding)


# speedup vs baseline: 1.9934x; 1.5396x over previous
import functools

import jax
import jax.numpy as jnp
from jax import lax
from jax.experimental import pallas as pl
from jax.experimental.pallas import tpu as pltpu
from jax.experimental.pallas import tpu_sc as plsc

_B = 16384
_D = 16
_NC = 2
_NS = 16
_NW = _NC * _NS
_BPW = _B // _NW
_NIDX = 3 * _BPW
_NPH = 32
_EPP = _NIDX // _NPH
_TPP = _BPW // _NPH

_LN2 = 0.6931471805599453
_SQRT2 = 1.4142135623730951


def _log16(x):
    xi = lax.bitcast_convert_type(x, jnp.int32)
    e = jnp.right_shift(xi, 23) - 127
    m = lax.bitcast_convert_type(
        jnp.bitwise_or(jnp.bitwise_and(xi, 0x007FFFFF), 0x3F800000), jnp.float32)
    big = m > _SQRT2
    m = jnp.where(big, m * 0.5, m)
    ef = e.astype(jnp.float32) + jnp.where(big, 1.0, 0.0)
    z = (m - 1.0) / (m + 1.0)
    z2 = z * z
    p = z * (2.0 + z2 * (0.66666667 + z2 * (0.4 + z2 * 0.28571429)))
    return ef * _LN2 + p


_mesh = plsc.VectorSubcoreMesh(core_axis_name="c", subcore_axis_name="s")


@functools.partial(
    pl.kernel,
    mesh=_mesh,
    compiler_params=pltpu.CompilerParams(
        needs_layout_passes=False, use_tc_tiling_on_sc=True),
    out_type=jax.ShapeDtypeStruct((_B,), jnp.float32),
    scratch_types=[
        pltpu.VMEM((_NIDX,), jnp.int32),
        pltpu.VMEM((_EPP, 8, _D), jnp.float32),
        pltpu.VMEM((_EPP, 8, _D), jnp.float32),
        pltpu.VMEM((_BPW,), jnp.float32),
        pltpu.SemaphoreType.DMA,
        pltpu.SemaphoreType.DMA,
    ],
)
def _tste_sc(idx_hbm, table_hbm, out_hbm, idx_v, rows_a, rows_b, out_v,
             sem0, sem1):
    wid = lax.axis_index("s") * _NC + lax.axis_index("c")

    pltpu.sync_copy(idx_hbm.at[wid], idx_v)

    lane = lax.iota(jnp.int32, 16)

    def fire(p, buf, sem):
        for g in range(_EPP // 16):
            vec = idx_v[pl.ds(p * _EPP + g * 16, 16)]
            for i in range(16):
                t = vec[i]
                part = jnp.where(t >= 500000, 1, 0)
                tb = pl.multiple_of(
                    jnp.right_shift(t - part * 500000, 3) * 8, 8)
                pltpu.async_copy(
                    table_hbm.at[part, pl.ds(tb, 8), :],
                    buf.at[g * 16 + i], sem)

    def drain(buf, sem):
        pltpu.make_async_copy(
            table_hbm.at[0, pl.ds(0, _EPP * 8), :].reshape(_EPP, 8, _D),
            buf, sem).wait()

    def compute(p, buf):
        e_h = p * _EPP + 3 * lane
        th = plsc.load_gather(idx_v, [e_h])
        tw = plsc.load_gather(idx_v, [e_h + 1])
        tl = plsc.load_gather(idx_v, [e_h + 2])
        sub_h = jnp.bitwise_and(th, 7)
        sub_w = jnp.bitwise_and(tw, 7)
        sub_l = jnp.bitwise_and(tl, 7)
        l_h = 3 * lane
        accw = jnp.zeros((16,), jnp.float32)
        accl = jnp.zeros((16,), jnp.float32)
        for d in range(_D):
            dv = jnp.full((16,), d, jnp.int32)
            hd = plsc.load_gather(buf, [l_h, sub_h, dv])
            wd = plsc.load_gather(buf, [l_h + 1, sub_w, dv])
            ld = plsc.load_gather(buf, [l_h + 2, sub_l, dv])
            dw = hd - wd
            dl = hd - ld
            accw = accw + dw * dw
            accl = accl + dl * dl
        x = 1.0 + (1.0 + accw) / (1.0 + accl)
        out_v[pl.ds(p * _TPP, 16)] = _log16(x)

    fire(0, rows_a, sem0)
    fire(1, rows_b, sem1)

    def pair(j, carry):
        p0 = 2 * j

        drain(rows_a, sem0)
        compute(p0, rows_a)

        @pl.when(p0 + 2 < _NPH)
        def _():
            fire(p0 + 2, rows_a, sem0)

        drain(rows_b, sem1)
        compute(p0 + 1, rows_b)

        @pl.when(p0 + 3 < _NPH)
        def _():
            fire(p0 + 3, rows_b, sem1)

        return carry

    lax.fori_loop(0, _NPH // 2, pair, 0)

    base = pl.multiple_of(wid * _BPW, 8)
    pltpu.sync_copy(out_v, out_hbm.at[pl.ds(base, _BPW)])


def kernel(h_w_l, embedding):
    idx = h_w_l.reshape(_NW, _NIDX)
    return _tste_sc(idx, embedding.reshape(2, 500000, _D))
